# baseline (speedup 1.0000x reference)
import functools

import jax
import jax.numpy as jnp
from jax import lax
from jax.experimental import pallas as pl
from jax.experimental.pallas import tpu as pltpu
from jax.experimental.pallas import tpu_sc as plsc

NC = 2
NS = 16
L = 16
NW = NC * NS

B = 16384
D = 128
G = 64
PER_W = B // NW
NCHUNK = PER_W // G


def _body(idx_hbm, uemb_hbm, memb_hbm, ubias_hbm, mbias_hbm,
          out_hbm,
          islab_v, uidx_v, midx_v, urows_v, mrows_v, ubias_v, mbias_v,
          p_v, out_v, sem0, sem1):
  wid = lax.axis_index("s") * NC + lax.axis_index("c")
  base = wid * PER_W

  iot = lax.iota(jnp.int32, L)
  zero = jnp.zeros((L,), jnp.int32)
  one = jnp.full((L,), 1, jnp.int32)
  sems = (sem0, sem1)

  pltpu.sync_copy(idx_hbm.at[pl.ds(base, PER_W), :], islab_v)
  for j in range(NCHUNK):
    for g in range(G // L):
      rows = iot + (j * G + g * L)
      uidx_v[j, pl.ds(g * L, L)] = plsc.load_gather(islab_v, [rows, zero])
      midx_v[j, pl.ds(g * L, L)] = plsc.load_gather(islab_v, [rows, one])

  def launch(j, b):
    sem = sems[b]
    pltpu.async_copy(uemb_hbm.at[uidx_v.at[j]], urows_v.at[b], sem)
    pltpu.async_copy(memb_hbm.at[midx_v.at[j]], mrows_v.at[b], sem)
    pltpu.async_copy(ubias_hbm.at[uidx_v.at[j]], ubias_v.at[b], sem)
    pltpu.async_copy(mbias_hbm.at[midx_v.at[j]], mbias_v.at[b], sem)

  def drain(j, b):
    sem = sems[b]
    pltpu.make_async_copy(uemb_hbm.at[uidx_v.at[j]], urows_v.at[b], sem).wait()
    pltpu.make_async_copy(memb_hbm.at[midx_v.at[j]], mrows_v.at[b], sem).wait()
    pltpu.make_async_copy(ubias_hbm.at[uidx_v.at[j]], ubias_v.at[b], sem).wait()
    pltpu.make_async_copy(mbias_hbm.at[midx_v.at[j]], mbias_v.at[b], sem).wait()

  def compute(j, b):
    bfull = jnp.full((L,), b, jnp.int32)
    for g in range(G // L):
      for i in range(L):
        row = g * L + i
        acc = urows_v[b, row, pl.ds(0, L)] * mrows_v[b, row, pl.ds(0, L)]
        for k in range(1, D // L):
          acc = acc + urows_v[b, row, pl.ds(k * L, L)] * mrows_v[b, row, pl.ds(k * L, L)]
        p_v[pl.ds(i * L, L)] = acc
      cols = iot * L
      tot = plsc.load_gather(p_v, [cols])
      for c in range(1, L):
        tot = tot + plsc.load_gather(p_v, [cols + c])
      brow = iot + g * L
      ub = plsc.load_gather(ubias_v, [bfull, brow, zero])
      mb = plsc.load_gather(mbias_v, [bfull, brow, zero])
      x = tot + ub + mb
      y = 5.0 / (1.0 + jnp.exp(-x))
      out_v[pl.ds(j * G + g * L, L)] = y

  launch(0, 0)

  def pair_body(t, carry):
    j0 = 2 * t
    j1 = j0 + 1
    launch(j1, 1)
    drain(j0, 0)
    compute(j0, 0)

    @pl.when(j1 + 1 < NCHUNK)
    def _():
      launch(j1 + 1, 0)

    drain(j1, 1)
    compute(j1, 1)
    return carry

  lax.fori_loop(0, NCHUNK // 2, pair_body, 0, unroll=False)
  pltpu.sync_copy(out_v, out_hbm.at[pl.ds(base, PER_W)])


@functools.partial(jax.jit, donate_argnums=())
def _run(idx, uemb, memb, ubias, mbias):
  mesh = plsc.VectorSubcoreMesh(core_axis_name="c", subcore_axis_name="s",
                                num_cores=NC, num_subcores=NS)
  fn = pl.kernel(
      _body,
      out_type=jax.ShapeDtypeStruct((B,), jnp.float32),
      mesh=mesh,
      compiler_params=pltpu.CompilerParams(needs_layout_passes=False,
                                           use_tc_tiling_on_sc=False),
      scratch_types=[
          pltpu.VMEM((PER_W, 2), jnp.int32),
          pltpu.VMEM((NCHUNK, G), jnp.int32),
          pltpu.VMEM((NCHUNK, G), jnp.int32),
          pltpu.VMEM((2, G, D), jnp.float32),
          pltpu.VMEM((2, G, D), jnp.float32),
          pltpu.VMEM((2, G, 1), jnp.float32),
          pltpu.VMEM((2, G, 1), jnp.float32),
          pltpu.VMEM((L * L,), jnp.float32),
          pltpu.VMEM((PER_W,), jnp.float32),
          pltpu.SemaphoreType.DMA,
          pltpu.SemaphoreType.DMA,
      ],
  )
  return fn(idx, uemb, memb, ubias, mbias)


def kernel(inputs, user_emb, user_bias, movie_emb, movie_bias):
  out = _run(inputs.astype(jnp.int32), user_emb, movie_emb,
             user_bias, movie_bias)
  return out.reshape(B, 1)


# speedup vs baseline: 6.1008x; 6.1008x over previous
import functools

import jax
import jax.numpy as jnp
from jax import lax
from jax.experimental import pallas as pl
from jax.experimental.pallas import tpu as pltpu
from jax.experimental.pallas import tpu_sc as plsc

NC = 2
NS = 16
L = 16
NW = NC * NS

B = 16384
D = 128
G = 64
PER_W = B // NW
NCHUNK = PER_W // G
BIDX = 128


def _body(uidx_hbm, midx_hbm, uemb_hbm, memb_hbm, ubias_hbm, mbias_hbm,
          out_hbm,
          uidx_v, midx_v, urows_v, mrows_v, ubias_v, mbias_v,
          p_v, out_v, sem0, sem1, semb):
  wid = lax.axis_index("s") * NC + lax.axis_index("c")
  base = wid * PER_W

  iot = lax.iota(jnp.int32, L)
  sems = (sem0, sem1)

  pltpu.sync_copy(uidx_hbm.at[pl.ds(base, PER_W)], uidx_v)
  pltpu.sync_copy(midx_hbm.at[pl.ds(base, PER_W)], midx_v)

  for q in range(PER_W // BIDX):
    sl = pl.ds(q * BIDX, BIDX)
    pltpu.async_copy(ubias_hbm.at[uidx_v.at[sl]], ubias_v.at[sl], semb)
    pltpu.async_copy(mbias_hbm.at[midx_v.at[sl]], mbias_v.at[sl], semb)

  def launch(j, b):
    sem = sems[b]
    sl = pl.ds(j * G, G)
    pltpu.async_copy(uemb_hbm.at[uidx_v.at[sl]], urows_v.at[b], sem)
    pltpu.async_copy(memb_hbm.at[midx_v.at[sl]], mrows_v.at[b], sem)

  def drain(j, b):
    sem = sems[b]
    sl = pl.ds(j * G, G)
    pltpu.make_async_copy(uemb_hbm.at[uidx_v.at[sl]], urows_v.at[b], sem).wait()
    pltpu.make_async_copy(memb_hbm.at[midx_v.at[sl]], mrows_v.at[b], sem).wait()

  def compute(j, b):
    def group_body(g, carry):
      for i in range(L):
        ro = g * L + i
        ts = [urows_v[b, ro, pl.ds(k * L, L)] * mrows_v[b, ro, pl.ds(k * L, L)]
              for k in range(D // L)]
        s0 = ts[0] + ts[1]
        s1 = ts[2] + ts[3]
        s2 = ts[4] + ts[5]
        s3 = ts[6] + ts[7]
        p_v[pl.ds(i * L, L)] = (s0 + s1) + (s2 + s3)
      cols = iot * L
      tot = plsc.load_gather(p_v, [cols])
      for c in range(1, L):
        tot = tot + plsc.load_gather(p_v, [cols + c])
      off = j * G + g * L
      x = tot + ubias_v[pl.ds(off, L)] + mbias_v[pl.ds(off, L)]
      y = 5.0 / (1.0 + jnp.exp(-x))
      out_v[pl.ds(off, L)] = y
      return carry

    lax.fori_loop(0, G // L, group_body, 0, unroll=False)

  launch(0, 0)

  for q in range(PER_W // BIDX):
    sl = pl.ds(q * BIDX, BIDX)
    pltpu.make_async_copy(ubias_hbm.at[uidx_v.at[sl]], ubias_v.at[sl], semb).wait()
    pltpu.make_async_copy(mbias_hbm.at[midx_v.at[sl]], mbias_v.at[sl], semb).wait()

  def pair_body(t, carry):
    j0 = 2 * t
    j1 = j0 + 1
    launch(j1, 1)
    drain(j0, 0)
    compute(j0, 0)

    @pl.when(j1 + 1 < NCHUNK)
    def _():
      launch(j1 + 1, 0)

    drain(j1, 1)
    compute(j1, 1)
    return carry

  lax.fori_loop(0, NCHUNK // 2, pair_body, 0, unroll=False)
  pltpu.sync_copy(out_v, out_hbm.at[pl.ds(base, PER_W)])


@functools.partial(jax.jit, donate_argnums=())
def _run(uidx, midx, uemb, memb, ubias, mbias):
  mesh = plsc.VectorSubcoreMesh(core_axis_name="c", subcore_axis_name="s",
                                num_cores=NC, num_subcores=NS)
  fn = pl.kernel(
      _body,
      out_type=jax.ShapeDtypeStruct((B,), jnp.float32),
      mesh=mesh,
      compiler_params=pltpu.CompilerParams(needs_layout_passes=False),
      scratch_types=[
          pltpu.VMEM((PER_W,), jnp.int32),
          pltpu.VMEM((PER_W,), jnp.int32),
          pltpu.VMEM((2, G, D), jnp.float32),
          pltpu.VMEM((2, G, D), jnp.float32),
          pltpu.VMEM((PER_W,), jnp.float32),
          pltpu.VMEM((PER_W,), jnp.float32),
          pltpu.VMEM((L * L,), jnp.float32),
          pltpu.VMEM((PER_W,), jnp.float32),
          pltpu.SemaphoreType.DMA,
          pltpu.SemaphoreType.DMA,
          pltpu.SemaphoreType.DMA,
      ],
  )
  return fn(uidx, midx, uemb, memb, ubias, mbias)


def kernel(inputs, user_emb, user_bias, movie_emb, movie_bias):
  idx = inputs.astype(jnp.int32)
  out = _run(idx[:, 0], idx[:, 1], user_emb, movie_emb,
             user_bias.reshape(-1), movie_bias.reshape(-1))
  return out.reshape(B, 1)


# speedup vs baseline: 6.1593x; 1.0096x over previous
import functools

import jax
import jax.numpy as jnp
from jax import lax
from jax.experimental import pallas as pl
from jax.experimental.pallas import tpu as pltpu
from jax.experimental.pallas import tpu_sc as plsc

NC = 2
NS = 16
L = 16
NW = NC * NS

B = 16384
D = 128
G = 64
PER_W = B // NW
NCHUNK = PER_W // G
BIDX = 128


def _body(uidx_hbm, midx_hbm, uemb_hbm, memb_hbm, ubias_hbm, mbias_hbm,
          out_hbm,
          uidx_v, midx_v, urows_v, mrows_v, ubias_v, mbias_v,
          p_v, out_v, sem0, sem1, semb):
  wid = lax.axis_index("s") * NC + lax.axis_index("c")
  base = wid * PER_W

  iot = lax.iota(jnp.int32, L)
  sems = (sem0, sem1)

  pltpu.sync_copy(uidx_hbm.at[pl.ds(base, PER_W)], uidx_v)
  pltpu.sync_copy(midx_hbm.at[pl.ds(base, PER_W)], midx_v)

  for q in range(PER_W // BIDX):
    sl = pl.ds(q * BIDX, BIDX)
    pltpu.async_copy(ubias_hbm.at[uidx_v.at[sl]], ubias_v.at[sl], semb)
    pltpu.async_copy(mbias_hbm.at[midx_v.at[sl]], mbias_v.at[sl], semb)

  def launch(j, b):
    sem = sems[b]
    sl = pl.ds(j * G, G)
    pltpu.async_copy(uemb_hbm.at[uidx_v.at[sl]], urows_v.at[b], sem)
    pltpu.async_copy(memb_hbm.at[midx_v.at[sl]], mrows_v.at[b], sem)

  def drain(j, b):
    sem = sems[b]
    sl = pl.ds(j * G, G)
    pltpu.make_async_copy(uemb_hbm.at[uidx_v.at[sl]], urows_v.at[b], sem).wait()
    pltpu.make_async_copy(memb_hbm.at[midx_v.at[sl]], mrows_v.at[b], sem).wait()

  def compute(j, b):
    @plsc.parallel_loop(0, G // L)
    def group_body(g):
      pbase = g * (L * L)
      for i in range(L):
        ro = g * L + i
        ts = [urows_v[b, ro, pl.ds(k * L, L)] * mrows_v[b, ro, pl.ds(k * L, L)]
              for k in range(D // L)]
        s0 = ts[0] + ts[1]
        s1 = ts[2] + ts[3]
        s2 = ts[4] + ts[5]
        s3 = ts[6] + ts[7]
        p_v[pl.ds(pbase + i * L, L)] = (s0 + s1) + (s2 + s3)
      cols = iot * L + pbase
      tot = plsc.load_gather(p_v, [cols])
      for c in range(1, L):
        tot = tot + plsc.load_gather(p_v, [cols + c])
      off = j * G + g * L
      x = tot + ubias_v[pl.ds(off, L)] + mbias_v[pl.ds(off, L)]
      y = 5.0 / (1.0 + jnp.exp(-x))
      out_v[pl.ds(off, L)] = y

  launch(0, 0)

  for q in range(PER_W // BIDX):
    sl = pl.ds(q * BIDX, BIDX)
    pltpu.make_async_copy(ubias_hbm.at[uidx_v.at[sl]], ubias_v.at[sl], semb).wait()
    pltpu.make_async_copy(mbias_hbm.at[midx_v.at[sl]], mbias_v.at[sl], semb).wait()

  def pair_body(t, carry):
    j0 = 2 * t
    j1 = j0 + 1
    launch(j1, 1)
    drain(j0, 0)
    compute(j0, 0)

    @pl.when(j1 + 1 < NCHUNK)
    def _():
      launch(j1 + 1, 0)

    drain(j1, 1)
    compute(j1, 1)
    return carry

  lax.fori_loop(0, NCHUNK // 2, pair_body, 0, unroll=False)
  pltpu.sync_copy(out_v, out_hbm.at[pl.ds(base, PER_W)])


@functools.partial(jax.jit, donate_argnums=())
def _run(uidx, midx, uemb, memb, ubias, mbias):
  mesh = plsc.VectorSubcoreMesh(core_axis_name="c", subcore_axis_name="s",
                                num_cores=NC, num_subcores=NS)
  fn = pl.kernel(
      _body,
      out_type=jax.ShapeDtypeStruct((B,), jnp.float32),
      mesh=mesh,
      compiler_params=pltpu.CompilerParams(needs_layout_passes=False),
      scratch_types=[
          pltpu.VMEM((PER_W,), jnp.int32),
          pltpu.VMEM((PER_W,), jnp.int32),
          pltpu.VMEM((2, G, D), jnp.float32),
          pltpu.VMEM((2, G, D), jnp.float32),
          pltpu.VMEM((PER_W,), jnp.float32),
          pltpu.VMEM((PER_W,), jnp.float32),
          pltpu.VMEM(((G // L) * L * L,), jnp.float32),
          pltpu.VMEM((PER_W,), jnp.float32),
          pltpu.SemaphoreType.DMA,
          pltpu.SemaphoreType.DMA,
          pltpu.SemaphoreType.DMA,
      ],
  )
  return fn(uidx, midx, uemb, memb, ubias, mbias)


def kernel(inputs, user_emb, user_bias, movie_emb, movie_bias):
  idx = inputs.astype(jnp.int32)
  out = _run(idx[:, 0], idx[:, 1], user_emb, movie_emb,
             user_bias.reshape(-1), movie_bias.reshape(-1))
  return out.reshape(B, 1)


# speedup vs baseline: 6.2208x; 1.0100x over previous
import functools

import jax
import jax.numpy as jnp
from jax import lax
from jax.experimental import pallas as pl
from jax.experimental.pallas import tpu as pltpu
from jax.experimental.pallas import tpu_sc as plsc

NC = 2
NS = 16
L = 16
NW = NC * NS

B = 16384
D = 128
G = 64
PER_W = B // NW
NCHUNK = PER_W // G
BIDX = 128


def _body(uidx_hbm, midx_hbm, uemb_hbm, memb_hbm, ubias_hbm, mbias_hbm,
          out_hbm,
          uidx_v, midx_v, urows_v, mrows_v, ubias_v, mbias_v,
          p_v, out_v, sem0, sem1, semb):
  wid = lax.axis_index("s") * NC + lax.axis_index("c")
  base = wid * PER_W

  iot = lax.iota(jnp.int32, L)
  sems = (sem0, sem1)

  pltpu.sync_copy(uidx_hbm.at[pl.ds(base, PER_W)], uidx_v)
  pltpu.sync_copy(midx_hbm.at[pl.ds(base, PER_W)], midx_v)

  for q in range(PER_W // BIDX):
    sl = pl.ds(q * BIDX, BIDX)
    pltpu.async_copy(ubias_hbm.at[uidx_v.at[sl]], ubias_v.at[sl], semb)
    pltpu.async_copy(mbias_hbm.at[midx_v.at[sl]], mbias_v.at[sl], semb)

  def launch(j, b):
    sem = sems[b]
    sl = pl.ds(j * G, G)
    pltpu.async_copy(uemb_hbm.at[uidx_v.at[sl]], urows_v.at[b], sem)
    pltpu.async_copy(memb_hbm.at[midx_v.at[sl]], mrows_v.at[b], sem)

  def drain(j, b):
    sem = sems[b]
    sl = pl.ds(j * G, G)
    pltpu.make_async_copy(uemb_hbm.at[uidx_v.at[sl]], urows_v.at[b], sem).wait()
    pltpu.make_async_copy(memb_hbm.at[midx_v.at[sl]], mrows_v.at[b], sem).wait()

  def compute(j, b):
    @plsc.parallel_loop(0, G // L, unroll=2)
    def group_body(g):
      pbase = g * (L * L)
      for i in range(L):
        ro = g * L + i
        ts = [urows_v[b, ro, pl.ds(k * L, L)] * mrows_v[b, ro, pl.ds(k * L, L)]
              for k in range(D // L)]
        s0 = ts[0] + ts[1]
        s1 = ts[2] + ts[3]
        s2 = ts[4] + ts[5]
        s3 = ts[6] + ts[7]
        p_v[pl.ds(pbase + i * L, L)] = (s0 + s1) + (s2 + s3)
      cols = iot * L + pbase
      tot = plsc.load_gather(p_v, [cols])
      for c in range(1, L):
        tot = tot + plsc.load_gather(p_v, [cols + c])
      off = j * G + g * L
      x = tot + ubias_v[pl.ds(off, L)] + mbias_v[pl.ds(off, L)]
      y = 5.0 / (1.0 + jnp.exp(-x))
      out_v[pl.ds(off, L)] = y

  launch(0, 0)

  for q in range(PER_W // BIDX):
    sl = pl.ds(q * BIDX, BIDX)
    pltpu.make_async_copy(ubias_hbm.at[uidx_v.at[sl]], ubias_v.at[sl], semb).wait()
    pltpu.make_async_copy(mbias_hbm.at[midx_v.at[sl]], mbias_v.at[sl], semb).wait()

  def pair_body(t, carry):
    j0 = 2 * t
    j1 = j0 + 1
    launch(j1, 1)
    drain(j0, 0)
    compute(j0, 0)

    @pl.when(j1 + 1 < NCHUNK)
    def _():
      launch(j1 + 1, 0)

    drain(j1, 1)
    compute(j1, 1)
    return carry

  lax.fori_loop(0, NCHUNK // 2, pair_body, 0, unroll=False)
  pltpu.sync_copy(out_v, out_hbm.at[pl.ds(base, PER_W)])


@functools.partial(jax.jit, donate_argnums=())
def _run(uidx, midx, uemb, memb, ubias, mbias):
  mesh = plsc.VectorSubcoreMesh(core_axis_name="c", subcore_axis_name="s",
                                num_cores=NC, num_subcores=NS)
  fn = pl.kernel(
      _body,
      out_type=jax.ShapeDtypeStruct((B,), jnp.float32),
      mesh=mesh,
      compiler_params=pltpu.CompilerParams(needs_layout_passes=False),
      scratch_types=[
          pltpu.VMEM((PER_W,), jnp.int32),
          pltpu.VMEM((PER_W,), jnp.int32),
          pltpu.VMEM((2, G, D), jnp.float32),
          pltpu.VMEM((2, G, D), jnp.float32),
          pltpu.VMEM((PER_W,), jnp.float32),
          pltpu.VMEM((PER_W,), jnp.float32),
          pltpu.VMEM(((G // L) * L * L,), jnp.float32),
          pltpu.VMEM((PER_W,), jnp.float32),
          pltpu.SemaphoreType.DMA,
          pltpu.SemaphoreType.DMA,
          pltpu.SemaphoreType.DMA,
      ],
  )
  return fn(uidx, midx, uemb, memb, ubias, mbias)


def kernel(inputs, user_emb, user_bias, movie_emb, movie_bias):
  idx = inputs.astype(jnp.int32)
  out = _run(idx[:, 0], idx[:, 1], user_emb, movie_emb,
             user_bias.reshape(-1), movie_bias.reshape(-1))
  return out.reshape(B, 1)
